# Initial kernel scaffold; baseline (speedup 1.0000x reference)
import jax
import jax.numpy as jnp
from jax.experimental import pallas as pl


def kernel(x, table):
    raise NotImplementedError("write your pallas kernel here")


# speedup vs baseline: 1.8766x; 1.8766x over previous
import functools

import jax
import jax.numpy as jnp
from jax import lax
from jax.experimental import pallas as pl
from jax.experimental.pallas import tpu as pltpu
from jax.experimental.pallas import tpu_sc as plsc

BATCH = 16384
HIST = 50
EMBED_DIM = 64
NTOT = BATCH * HIST

NC = 2
NS = 16
NW = NC * NS
BPW = NTOT // NW

C = 128
NCH = BPW // C
NBUF = 8

_mesh = plsc.VectorSubcoreMesh(
    core_axis_name="c", subcore_axis_name="s", num_cores=NC, num_subcores=NS
)


@functools.partial(
    pl.kernel,
    out_type=jax.ShapeDtypeStruct((NTOT, EMBED_DIM), jnp.float32),
    mesh=_mesh,
    compiler_params=pltpu.CompilerParams(use_tc_tiling_on_sc=False),
    scratch_types=[
        pltpu.VMEM((NCH, C), jnp.int32),
        pltpu.VMEM((NBUF, C, EMBED_DIM), jnp.float32),
        pltpu.SemaphoreType.DMA((NBUF,)),
        pltpu.SemaphoreType.DMA((NBUF,)),
    ],
)
def _embed_gather(x_hbm, table_hbm, out_hbm, idx_v, rows_v, gsem, osem):
    wid = lax.axis_index("s") * NC + lax.axis_index("c")
    chunk0 = wid * NCH

    pltpu.sync_copy(x_hbm.at[pl.ds(chunk0, NCH)], idx_v)

    def gather_start(ch, b):
        pltpu.make_async_copy(
            table_hbm.at[idx_v.at[ch]], rows_v.at[b], gsem.at[b]
        ).start()

    def gather_wait(ch, b):
        pltpu.make_async_copy(
            table_hbm.at[idx_v.at[ch]], rows_v.at[b], gsem.at[b]
        ).wait()

    def out_start(ch, b):
        pltpu.make_async_copy(
            rows_v.at[b], out_hbm.at[pl.ds((chunk0 + ch) * C, C)], osem.at[b]
        ).start()

    def out_wait(ch, b):
        pltpu.make_async_copy(
            rows_v.at[b], out_hbm.at[pl.ds((chunk0 + ch) * C, C)], osem.at[b]
        ).wait()

    for b in range(NBUF):
        gather_start(b, b)

    @pl.loop(0, NCH - NBUF, step=NBUF)
    def _main(g):
        for b in range(NBUF):
            ch = g + b
            gather_wait(ch, b)
            out_start(ch, b)
            out_wait(ch, b)
            gather_start(ch + NBUF, b)

    for b in range(NBUF):
        ch = NCH - NBUF + b
        gather_wait(ch, b)
        out_start(ch, b)
    for b in range(NBUF):
        ch = NCH - NBUF + b
        out_wait(ch, b)


def kernel(x, table):
    x2d = x.reshape(NW * NCH, C).astype(jnp.int32)
    out = _embed_gather(x2d, table)
    return out.reshape(BATCH, HIST, EMBED_DIM)


# speedup vs baseline: 1.8793x; 1.0014x over previous
import functools

import jax
import jax.numpy as jnp
from jax import lax
from jax.experimental import pallas as pl
from jax.experimental.pallas import tpu as pltpu
from jax.experimental.pallas import tpu_sc as plsc

BATCH = 16384
HIST = 50
EMBED_DIM = 64
NTOT = BATCH * HIST

NC = 2
NS = 16
NW = NC * NS
BPW = NTOT // NW

C = 128
NCH = BPW // C
NBUF = 10
K = 7

_mesh = plsc.VectorSubcoreMesh(
    core_axis_name="c", subcore_axis_name="s", num_cores=NC, num_subcores=NS
)


@functools.partial(
    pl.kernel,
    out_type=jax.ShapeDtypeStruct((NTOT, EMBED_DIM), jnp.float32),
    mesh=_mesh,
    compiler_params=pltpu.CompilerParams(use_tc_tiling_on_sc=False),
    scratch_types=[
        pltpu.VMEM((NCH, C), jnp.int32),
        pltpu.VMEM((NBUF, C, EMBED_DIM), jnp.float32),
        pltpu.SemaphoreType.DMA((NBUF,)),
        pltpu.SemaphoreType.DMA((NBUF,)),
    ],
)
def _embed_gather(x_hbm, table_hbm, out_hbm, idx_v, rows_v, gsem, osem):
    wid = lax.axis_index("s") * NC + lax.axis_index("c")
    chunk0 = wid * NCH

    pltpu.sync_copy(x_hbm.at[pl.ds(chunk0, NCH)], idx_v)

    def gather_start(ch, b):
        pltpu.make_async_copy(
            table_hbm.at[idx_v.at[ch]], rows_v.at[b], gsem.at[b]
        ).start()

    def gather_wait(ch, b):
        pltpu.make_async_copy(
            table_hbm.at[idx_v.at[ch]], rows_v.at[b], gsem.at[b]
        ).wait()

    def out_start(ch, b):
        pltpu.make_async_copy(
            rows_v.at[b], out_hbm.at[pl.ds((chunk0 + ch) * C, C)], osem.at[b]
        ).start()

    def out_wait(ch, b):
        pltpu.make_async_copy(
            rows_v.at[b], out_hbm.at[pl.ds((chunk0 + ch) * C, C)], osem.at[b]
        ).wait()

    for i in range(NBUF):
        gather_start(i, i)
        if i >= K:
            gather_wait(i - K, i - K)
            out_start(i - K, i - K)

    @pl.loop(NBUF, NCH, step=NBUF)
    def _main(g):
        for j in range(NBUF):
            i = g + j
            out_wait(i - NBUF, j)
            gather_start(i, j)
            gather_wait(i - K, (j - K) % NBUF)
            out_start(i - K, (j - K) % NBUF)

    for i in range(NCH, NCH + K):
        gather_wait(i - K, (i - K) % NBUF)
        out_start(i - K, (i - K) % NBUF)
    for ch in range(NCH - NBUF, NCH):
        out_wait(ch, ch % NBUF)


def kernel(x, table):
    x2d = x.reshape(NW * NCH, C).astype(jnp.int32)
    out = _embed_gather(x2d, table)
    return out.reshape(BATCH, HIST, EMBED_DIM)
